# Initial kernel scaffold; baseline (speedup 1.0000x reference)
import jax
import jax.numpy as jnp
from jax.experimental import pallas as pl


def kernel(x, edge_index, w1_0, b1_0, w2_0, b2_0, W1, B1, W2, B2, w_out, b_out):
    raise NotImplementedError("write your pallas kernel here")


# speedup vs baseline: 6.0871x; 6.0871x over previous
import functools

import jax
import jax.numpy as jnp
from jax import lax
from jax.experimental import pallas as pl
from jax.experimental.pallas import tpu as pltpu
from jax.experimental.pallas import tpu_sc as plsc

N = 10000
E = 640000
H = 128
NC = 2
NS = 16
CHUNK = 80
ITERS = E // (NC * NS * CHUNK)


def _sc_agg_body(v_hbm, src_hbm, dst_hbm, zero_hbm, out_hbm,
                 idx_v, dst_v, rows_v, acc, sem):
    c = lax.axis_index("c")
    s = lax.axis_index("s")

    @pl.when(s == 0)
    def _init():
        pltpu.sync_copy(zero_hbm, acc)

    plsc.subcore_barrier()

    base = (c * NS + s) * (CHUNK * ITERS)

    def step(i, carry):
        off = pl.multiple_of(base + i * CHUNK, 8)
        pltpu.sync_copy(src_hbm.at[pl.ds(off, CHUNK)], idx_v)
        pltpu.sync_copy(dst_hbm.at[pl.ds(off, CHUNK)], dst_v)
        pltpu.async_copy(v_hbm.at[idx_v], rows_v, sem).wait()
        pltpu.sync_copy(rows_v, acc.at[dst_v], add=True)
        return carry

    lax.fori_loop(0, ITERS, step, 0)
    plsc.subcore_barrier()

    @pl.when(s < NS - 1)
    def _copy_main():
        r0 = s * 624
        pltpu.sync_copy(acc.at[pl.ds(r0, 624)], out_hbm.at[c, pl.ds(r0, 624)])

    @pl.when(s == NS - 1)
    def _copy_tail():
        r0 = (NS - 1) * 624
        pltpu.sync_copy(acc.at[pl.ds(r0, N - r0)],
                        out_hbm.at[c, pl.ds(r0, N - r0)])


@functools.cache
def _make_sc_agg():
    @functools.partial(
        pl.kernel,
        out_type=jax.ShapeDtypeStruct((NC, N, H), jnp.float32),
        mesh=plsc.VectorSubcoreMesh(core_axis_name="c", subcore_axis_name="s",
                                    num_cores=NC, num_subcores=NS),
        scratch_types=[
            pltpu.VMEM((CHUNK,), jnp.int32),
            pltpu.VMEM((CHUNK,), jnp.int32),
            pltpu.VMEM((CHUNK, H), jnp.float32),
            pltpu.VMEM_SHARED((N, H), jnp.float32),
            pltpu.SemaphoreType.DMA,
        ],
    )
    def sc_agg(*args):
        _sc_agg_body(*args)

    return sc_agg


def _sc_agg(v, src, dst, zero):
    return _make_sc_agg()(v, src, dst, zero)


BM = 1000


def _mm_body(x_ref, w_ref, b_ref, o_ref):
    o_ref[...] = jnp.dot(x_ref[...], w_ref[...],
                         preferred_element_type=jnp.float32) + b_ref[...]


def _mm(x, w, b):
    return pl.pallas_call(
        _mm_body,
        grid=(N // BM,),
        in_specs=[
            pl.BlockSpec((BM, H), lambda i: (i, 0)),
            pl.BlockSpec((H, H), lambda i: (0, 0)),
            pl.BlockSpec((1, H), lambda i: (0, 0)),
        ],
        out_specs=pl.BlockSpec((BM, H), lambda i: (i, 0)),
        out_shape=jax.ShapeDtypeStruct((N, H), jnp.float32),
    )(x, w, b)


def _mlp_body(h_ref, p0_ref, p1_ref, w1_ref, b1_ref, w2_ref, b2_ref, o_ref):
    z = h_ref[...] + (p0_ref[...] + p1_ref[...])
    t = jnp.dot(z, w1_ref[...], preferred_element_type=jnp.float32)
    t = jnp.maximum(t + b1_ref[...], 0.0)
    u = jnp.dot(t, w2_ref[...], preferred_element_type=jnp.float32)
    o_ref[...] = jnp.maximum(u + b2_ref[...], 0.0)


def _mlp(h, p0, p1, w1, b1, w2, b2):
    row = lambda i: (i, 0)
    fixed = lambda i: (0, 0)
    return pl.pallas_call(
        _mlp_body,
        grid=(N // BM,),
        in_specs=[
            pl.BlockSpec((BM, H), row),
            pl.BlockSpec((BM, H), row),
            pl.BlockSpec((BM, H), row),
            pl.BlockSpec((H, H), fixed),
            pl.BlockSpec((1, H), fixed),
            pl.BlockSpec((H, H), fixed),
            pl.BlockSpec((1, H), fixed),
        ],
        out_specs=pl.BlockSpec((BM, H), row),
        out_shape=jax.ShapeDtypeStruct((N, H), jnp.float32),
    )(h, p0, p1, w1, b1, w2, b2)


def kernel(x, edge_index, w1_0, b1_0, w2_0, b2_0, W1, B1, W2, B2, w_out,
           b_out):
    src = edge_index[0]
    dst = edge_index[1]
    zero = jnp.zeros((N, H), jnp.float32)

    h = jnp.pad(x, ((0, 0), (0, H - x.shape[1])))
    w1p = jnp.pad(w1_0, ((0, H - w1_0.shape[0]), (0, 0)))

    w1s = [w1p] + [W1[i] for i in range(4)]
    b1s = [b1_0.reshape(1, H)] + [B1[i].reshape(1, H) for i in range(4)]
    w2s = [w2_0] + [W2[i] for i in range(4)]
    b2s = [b2_0.reshape(1, H)] + [B2[i].reshape(1, H) for i in range(4)]

    for j in range(5):
        P = _sc_agg(h, src, dst, zero)
        h = _mlp(h, P[0], P[1], w1s[j], b1s[j], w2s[j], b2s[j])

    wout_p = jnp.pad(w_out, ((0, 0), (0, H - 1)))
    bout_p = jnp.pad(b_out, (0, H - 1)).reshape(1, H)
    scores = _mm(h, wout_p, bout_p)
    return scores[:, 0]


# speedup vs baseline: 16.6177x; 2.7300x over previous
import functools

import jax
import jax.numpy as jnp
from jax import lax
from jax.experimental import pallas as pl
from jax.experimental.pallas import tpu as pltpu
from jax.experimental.pallas import tpu_sc as plsc

N = 10000
E = 640000
H = 128
NC = 2
NS = 16
CHUNK = 125
ITERS = E // (NC * NS * CHUNK)
NB = 4
DEPTH = 2


W = 8
NWIN = ITERS // W


def _sc_agg_body(v_hbm, src_hbm, dst_hbm, zero_hbm, out_hbm,
                 srcw, dstw, rows, isems_s, isems_d, gsems, ssems, acc):
    c = lax.axis_index("c")
    s = lax.axis_index("s")
    tb = (c * NS + s) * ITERS

    @pl.when(s < NS - 1)
    def _init_main():
        pltpu.sync_copy(zero_hbm.at[pl.ds(s * 624, 624)],
                        acc.at[pl.ds(s * 624, 624)])

    @pl.when(s == NS - 1)
    def _init_tail():
        t0 = (NS - 1) * 624
        pltpu.sync_copy(zero_hbm.at[pl.ds(t0, N - t0)],
                        acc.at[pl.ds(t0, N - t0)])

    def iload_start(wi, p):
        pltpu.async_copy(src_hbm.at[pl.ds(tb + wi * W, W)], srcw[p],
                         isems_s[p])
        pltpu.async_copy(dst_hbm.at[pl.ds(tb + wi * W, W)], dstw[p],
                         isems_d[p])

    def iload_wait(wi, p):
        pltpu.make_async_copy(src_hbm.at[pl.ds(tb + wi * W, W)], srcw[p],
                              isems_s[p]).wait()
        pltpu.make_async_copy(dst_hbm.at[pl.ds(tb + wi * W, W)], dstw[p],
                              isems_d[p]).wait()

    def g_start(k, p, b):
        pltpu.async_copy(v_hbm.at[srcw[p].at[k]], rows[b], gsems[b])

    def g_wait(k, p, b):
        pltpu.make_async_copy(v_hbm.at[srcw[p].at[k]], rows[b],
                              gsems[b]).wait()

    def s_start(k, p, b):
        pltpu.async_copy(rows[b], acc.at[dstw[p].at[k]], ssems[b], add=True)

    def s_wait(k, p, b):
        pltpu.make_async_copy(rows[b], acc.at[dstw[p].at[k]],
                              ssems[b]).wait()

    iload_start(0, 0)
    iload_wait(0, 0)
    plsc.subcore_barrier()
    g_start(0, 0, 0)

    def superwindow(g, carry):
        for half in range(2):
            wi = g * 2 + half
            p = half

            @pl.when(wi + 1 < NWIN)
            def _start_next_iload():
                iload_start(wi + 1, p ^ 1)

            for k in range(W):
                b = k % 2
                i = wi * W + k

                @pl.when(i >= 1)
                def _retire():
                    if k == 0:
                        s_wait(W - 1, p ^ 1, b ^ 1)
                    else:
                        s_wait(k - 1, p, b ^ 1)

                if k == W - 1:
                    @pl.when(wi + 1 < NWIN)
                    def _boundary():
                        iload_wait(wi + 1, p ^ 1)
                        g_start(0, p ^ 1, b ^ 1)
                else:
                    g_start(k + 1, p, b ^ 1)

                g_wait(k, p, b)
                s_start(k, p, b)
        return carry

    lax.fori_loop(0, NWIN // 2, superwindow, 0)
    s_wait(W - 1, 1, 1)
    plsc.subcore_barrier()

    @pl.when(s < NS - 1)
    def _copy_main():
        r0 = s * 624
        pltpu.sync_copy(acc.at[pl.ds(r0, 624)], out_hbm.at[c, pl.ds(r0, 624)])

    @pl.when(s == NS - 1)
    def _copy_tail():
        t0 = (NS - 1) * 624
        pltpu.sync_copy(acc.at[pl.ds(t0, N - t0)],
                        out_hbm.at[c, pl.ds(t0, N - t0)])


@functools.cache
def _make_sc_agg():
    @functools.partial(
        pl.kernel,
        out_type=jax.ShapeDtypeStruct((NC, N, H), jnp.float32),
        mesh=plsc.VectorSubcoreMesh(core_axis_name="c", subcore_axis_name="s",
                                    num_cores=NC, num_subcores=NS),
        scratch_types=[
            [pltpu.VMEM((W, CHUNK), jnp.int32) for _ in range(2)],
            [pltpu.VMEM((W, CHUNK), jnp.int32) for _ in range(2)],
            [pltpu.VMEM((CHUNK, H), jnp.float32) for _ in range(2)],
            [pltpu.SemaphoreType.DMA for _ in range(2)],
            [pltpu.SemaphoreType.DMA for _ in range(2)],
            [pltpu.SemaphoreType.DMA for _ in range(2)],
            [pltpu.SemaphoreType.DMA for _ in range(2)],
            pltpu.VMEM_SHARED((N, H), jnp.float32),
        ],
    )
    def sc_agg(*args):
        _sc_agg_body(*args)

    return sc_agg


def _sc_agg(v, src, dst, zero):
    src2 = src.reshape(E // CHUNK, CHUNK)
    dst2 = dst.reshape(E // CHUNK, CHUNK)
    return _make_sc_agg()(v, src2, dst2, zero)


BM = 1000


def _mm_body(x_ref, w_ref, b_ref, o_ref):
    o_ref[...] = jnp.dot(x_ref[...], w_ref[...],
                         preferred_element_type=jnp.float32) + b_ref[...]


def _mm(x, w, b):
    return pl.pallas_call(
        _mm_body,
        grid=(N // BM,),
        in_specs=[
            pl.BlockSpec((BM, H), lambda i: (i, 0)),
            pl.BlockSpec((H, H), lambda i: (0, 0)),
            pl.BlockSpec((1, H), lambda i: (0, 0)),
        ],
        out_specs=pl.BlockSpec((BM, H), lambda i: (i, 0)),
        out_shape=jax.ShapeDtypeStruct((N, H), jnp.float32),
    )(x, w, b)


def _mlp_body(h_ref, p0_ref, p1_ref, w1_ref, b1_ref, w2_ref, b2_ref, o_ref):
    z = h_ref[...] + (p0_ref[...] + p1_ref[...])
    t = jnp.dot(z, w1_ref[...], preferred_element_type=jnp.float32)
    t = jnp.maximum(t + b1_ref[...], 0.0)
    u = jnp.dot(t, w2_ref[...], preferred_element_type=jnp.float32)
    o_ref[...] = jnp.maximum(u + b2_ref[...], 0.0)


def _mlp(h, p0, p1, w1, b1, w2, b2):
    row = lambda i: (i, 0)
    fixed = lambda i: (0, 0)
    return pl.pallas_call(
        _mlp_body,
        grid=(N // BM,),
        in_specs=[
            pl.BlockSpec((BM, H), row),
            pl.BlockSpec((BM, H), row),
            pl.BlockSpec((BM, H), row),
            pl.BlockSpec((H, H), fixed),
            pl.BlockSpec((1, H), fixed),
            pl.BlockSpec((H, H), fixed),
            pl.BlockSpec((1, H), fixed),
        ],
        out_specs=pl.BlockSpec((BM, H), row),
        out_shape=jax.ShapeDtypeStruct((N, H), jnp.float32),
    )(h, p0, p1, w1, b1, w2, b2)


def kernel(x, edge_index, w1_0, b1_0, w2_0, b2_0, W1, B1, W2, B2, w_out,
           b_out):
    src = edge_index[0]
    dst = edge_index[1]
    zero = jnp.zeros((N, H), jnp.float32)

    h = jnp.pad(x, ((0, 0), (0, H - x.shape[1])))
    w1p = jnp.pad(w1_0, ((0, H - w1_0.shape[0]), (0, 0)))

    w1s = [w1p] + [W1[i] for i in range(4)]
    b1s = [b1_0.reshape(1, H)] + [B1[i].reshape(1, H) for i in range(4)]
    w2s = [w2_0] + [W2[i] for i in range(4)]
    b2s = [b2_0.reshape(1, H)] + [B2[i].reshape(1, H) for i in range(4)]

    for j in range(5):
        P = _sc_agg(h, src, dst, zero)
        h = _mlp(h, P[0], P[1], w1s[j], b1s[j], w2s[j], b2s[j])

    wout_p = jnp.pad(w_out, ((0, 0), (0, H - 1)))
    bout_p = jnp.pad(b_out, (0, H - 1)).reshape(1, H)
    scores = _mm(h, wout_p, bout_p)
    return scores[:, 0]


# speedup vs baseline: 16.8570x; 1.0144x over previous
import functools

import jax
import jax.numpy as jnp
from jax import lax
from jax.experimental import pallas as pl
from jax.experimental.pallas import tpu as pltpu
from jax.experimental.pallas import tpu_sc as plsc

N = 10000
E = 640000
H = 128
NC = 2
NS = 16
CHUNK = 125
ITERS = E // (NC * NS * CHUNK)
NB = 4
DEPTH = 2


W = 8
NWIN = ITERS // W


def _sc_agg_body(v_hbm, src_hbm, dst_hbm, zero_hbm, out_hbm,
                 srcw, dstw, rows, isems_s, isems_d, gsems, ssems, acc):
    c = lax.axis_index("c")
    s = lax.axis_index("s")
    tb = (c * NS + s) * ITERS

    r0i = s * 624
    t0i = (NS - 1) * 624
    main = s < NS - 1
    tail = s == NS - 1

    @pl.when(jnp.logical_and(main, c == 0))
    def _init_main_v():
        pltpu.sync_copy(v_hbm.at[pl.ds(r0i, 624)], acc.at[pl.ds(r0i, 624)])

    @pl.when(jnp.logical_and(main, c == 1))
    def _init_main_z():
        pltpu.sync_copy(zero_hbm.at[pl.ds(r0i, 624)], acc.at[pl.ds(r0i, 624)])

    @pl.when(jnp.logical_and(tail, c == 0))
    def _init_tail_v():
        pltpu.sync_copy(v_hbm.at[pl.ds(t0i, N - t0i)],
                        acc.at[pl.ds(t0i, N - t0i)])

    @pl.when(jnp.logical_and(tail, c == 1))
    def _init_tail_z():
        pltpu.sync_copy(zero_hbm.at[pl.ds(t0i, N - t0i)],
                        acc.at[pl.ds(t0i, N - t0i)])

    def iload_start(wi, p):
        pltpu.async_copy(src_hbm.at[pl.ds(tb + wi * W, W)], srcw[p],
                         isems_s[p])
        pltpu.async_copy(dst_hbm.at[pl.ds(tb + wi * W, W)], dstw[p],
                         isems_d[p])

    def iload_wait(wi, p):
        pltpu.make_async_copy(src_hbm.at[pl.ds(tb + wi * W, W)], srcw[p],
                              isems_s[p]).wait()
        pltpu.make_async_copy(dst_hbm.at[pl.ds(tb + wi * W, W)], dstw[p],
                              isems_d[p]).wait()

    def g_start(k, p, b):
        pltpu.async_copy(v_hbm.at[srcw[p].at[k]], rows[b], gsems[b])

    def g_wait(k, p, b):
        pltpu.make_async_copy(v_hbm.at[srcw[p].at[k]], rows[b],
                              gsems[b]).wait()

    def s_start(k, p, b):
        pltpu.async_copy(rows[b], acc.at[dstw[p].at[k]], ssems[b], add=True)

    def s_wait(k, p, b):
        pltpu.make_async_copy(rows[b], acc.at[dstw[p].at[k]],
                              ssems[b]).wait()

    iload_start(0, 0)
    iload_wait(0, 0)
    plsc.subcore_barrier()
    g_start(0, 0, 0)

    def superwindow(g, carry):
        for half in range(2):
            wi = g * 2 + half
            p = half

            @pl.when(wi + 1 < NWIN)
            def _start_next_iload():
                iload_start(wi + 1, p ^ 1)

            for k in range(W):
                b = k % 2
                i = wi * W + k

                @pl.when(i >= 1)
                def _retire():
                    if k == 0:
                        s_wait(W - 1, p ^ 1, b ^ 1)
                    else:
                        s_wait(k - 1, p, b ^ 1)

                if k == W - 1:
                    @pl.when(wi + 1 < NWIN)
                    def _boundary():
                        iload_wait(wi + 1, p ^ 1)
                        g_start(0, p ^ 1, b ^ 1)
                else:
                    g_start(k + 1, p, b ^ 1)

                g_wait(k, p, b)
                s_start(k, p, b)
        return carry

    lax.fori_loop(0, NWIN // 2, superwindow, 0)
    s_wait(W - 1, 1, 1)
    plsc.subcore_barrier()

    @pl.when(s < NS - 1)
    def _copy_main():
        r0 = s * 624
        pltpu.sync_copy(acc.at[pl.ds(r0, 624)], out_hbm.at[c, pl.ds(r0, 624)])

    @pl.when(s == NS - 1)
    def _copy_tail():
        t0 = (NS - 1) * 624
        pltpu.sync_copy(acc.at[pl.ds(t0, N - t0)],
                        out_hbm.at[c, pl.ds(t0, N - t0)])


@functools.cache
def _make_sc_agg():
    @functools.partial(
        pl.kernel,
        out_type=jax.ShapeDtypeStruct((NC, N, H), jnp.float32),
        mesh=plsc.VectorSubcoreMesh(core_axis_name="c", subcore_axis_name="s",
                                    num_cores=NC, num_subcores=NS),
        scratch_types=[
            [pltpu.VMEM((W, CHUNK), jnp.int32) for _ in range(2)],
            [pltpu.VMEM((W, CHUNK), jnp.int32) for _ in range(2)],
            [pltpu.VMEM((CHUNK, H), jnp.float32) for _ in range(2)],
            [pltpu.SemaphoreType.DMA for _ in range(2)],
            [pltpu.SemaphoreType.DMA for _ in range(2)],
            [pltpu.SemaphoreType.DMA for _ in range(2)],
            [pltpu.SemaphoreType.DMA for _ in range(2)],
            pltpu.VMEM_SHARED((N, H), jnp.float32),
        ],
    )
    def sc_agg(*args):
        _sc_agg_body(*args)

    return sc_agg


def _sc_agg(v, src, dst, zero):
    src2 = src.reshape(E // CHUNK, CHUNK)
    dst2 = dst.reshape(E // CHUNK, CHUNK)
    return _make_sc_agg()(v, src2, dst2, zero)


BM = 1000


def _mlp_body(p0_ref, p1_ref, w1_ref, b1_ref, w2_ref, b2_ref, o_ref):
    z = p0_ref[...] + p1_ref[...]
    t = jnp.dot(z, w1_ref[...], preferred_element_type=jnp.float32)
    t = jnp.maximum(t + b1_ref[...], 0.0)
    u = jnp.dot(t, w2_ref[...], preferred_element_type=jnp.float32)
    o_ref[...] = jnp.maximum(u + b2_ref[...], 0.0)


def _mlp_out_body(p0_ref, p1_ref, w1_ref, b1_ref, w2_ref, b2_ref, wo_ref,
                  bo_ref, o_ref):
    z = p0_ref[...] + p1_ref[...]
    t = jnp.dot(z, w1_ref[...], preferred_element_type=jnp.float32)
    t = jnp.maximum(t + b1_ref[...], 0.0)
    u = jnp.dot(t, w2_ref[...], preferred_element_type=jnp.float32)
    hh = jnp.maximum(u + b2_ref[...], 0.0)
    o_ref[...] = jnp.dot(hh, wo_ref[...],
                         preferred_element_type=jnp.float32) + bo_ref[...]


def _mlp(p0, p1, w1, b1, w2, b2):
    row = lambda i: (i, 0)
    fixed = lambda i: (0, 0)
    return pl.pallas_call(
        _mlp_body,
        grid=(N // BM,),
        in_specs=[
            pl.BlockSpec((BM, H), row),
            pl.BlockSpec((BM, H), row),
            pl.BlockSpec((H, H), fixed),
            pl.BlockSpec((1, H), fixed),
            pl.BlockSpec((H, H), fixed),
            pl.BlockSpec((1, H), fixed),
        ],
        out_specs=pl.BlockSpec((BM, H), row),
        out_shape=jax.ShapeDtypeStruct((N, H), jnp.float32),
    )(p0, p1, w1, b1, w2, b2)


def _mlp_out(p0, p1, w1, b1, w2, b2, wo, bo):
    row = lambda i: (i, 0)
    fixed = lambda i: (0, 0)
    return pl.pallas_call(
        _mlp_out_body,
        grid=(N // BM,),
        in_specs=[
            pl.BlockSpec((BM, H), row),
            pl.BlockSpec((BM, H), row),
            pl.BlockSpec((H, H), fixed),
            pl.BlockSpec((1, H), fixed),
            pl.BlockSpec((H, H), fixed),
            pl.BlockSpec((1, H), fixed),
            pl.BlockSpec((H, H), fixed),
            pl.BlockSpec((1, H), fixed),
        ],
        out_specs=pl.BlockSpec((BM, H), row),
        out_shape=jax.ShapeDtypeStruct((N, H), jnp.float32),
    )(p0, p1, w1, b1, w2, b2, wo, bo)


def kernel(x, edge_index, w1_0, b1_0, w2_0, b2_0, W1, B1, W2, B2, w_out,
           b_out):
    src = edge_index[0]
    dst = edge_index[1]
    zero = jnp.zeros((N, H), jnp.float32)

    h = jnp.pad(x, ((0, 0), (0, H - x.shape[1])))
    w1p = jnp.pad(w1_0, ((0, H - w1_0.shape[0]), (0, 0)))

    w1s = [w1p] + [W1[i] for i in range(4)]
    b1s = [b1_0.reshape(1, H)] + [B1[i].reshape(1, H) for i in range(4)]
    w2s = [w2_0] + [W2[i] for i in range(4)]
    b2s = [b2_0.reshape(1, H)] + [B2[i].reshape(1, H) for i in range(4)]

    for j in range(4):
        P = _sc_agg(h, src, dst, zero)
        h = _mlp(P[0], P[1], w1s[j], b1s[j], w2s[j], b2s[j])

    wout_p = jnp.pad(w_out, ((0, 0), (0, H - 1)))
    bout_p = jnp.pad(b_out, (0, H - 1)).reshape(1, H)
    P = _sc_agg(h, src, dst, zero)
    scores = _mlp_out(P[0], P[1], w1s[4], b1s[4], w2s[4], b2s[4],
                      wout_p, bout_p)
    return scores[:, 0]
